# baseline (speedup 1.0000x reference)
import functools

import jax
import jax.numpy as jnp
from jax import lax
from jax.experimental import pallas as pl
from jax.experimental.pallas import tpu as pltpu, tpu_sc as plsc

B = 4096
F = 26
V = 100001
D = 16
C = 4

NC = 2
NS = 16
NW = NC * NS
BPW = B // NW
EPW = BPW * F
SLOTS = F + 1
QPW = BPW * SLOTS
LANES = 16


def _body(tab_hbm, idx_hbm, cont_hbm, par_hbm, out_hbm,
          idxraw_v, cont_v, par_v, gidx_v, blk_v, dsem):
    wid = lax.axis_index("c") * NS + lax.axis_index("s")

    pltpu.sync_copy(idx_hbm.at[wid], idxraw_v)
    pltpu.sync_copy(cont_hbm.at[wid], cont_v)
    pltpu.sync_copy(par_hbm, par_v)

    iota = lax.iota(jnp.int32, LANES)

    def build_row(row, _):
        vSLOTS = jnp.full((LANES,), SLOTS, jnp.int32)
        for u in range(8):
            qv = (row * 8 + u) * LANES + iota
            bv = lax.div(qv, vSLOTS)
            sv = qv - bv * SLOTS
            pos = bv * F + jnp.minimum(sv, F - 1)
            raw = plsc.load_gather(idxraw_v, [pos])
            src = jnp.where(sv < F, raw + sv * V, 0)
            gidx_v[row, pl.ds(u * LANES, LANES)] = src
        return ()
    lax.fori_loop(0, SLOTS, build_row, (), unroll=False)

    def gather_fire(row, _):
        pltpu.async_copy(
            tab_hbm.at[gidx_v.at[row]],
            blk_v.at[pl.ds(row * BPW, BPW)],
            dsem,
        )
        return ()

    lax.fori_loop(0, SLOTS, gather_fire, (), unroll=False)

    def gather_drain(row, _):
        pltpu.make_async_copy(
            tab_hbm.at[gidx_v.at[row]],
            blk_v.at[pl.ds(row * BPW, BPW)],
            dsem,
        ).wait()
        return ()

    lax.fori_loop(0, SLOTS, gather_drain, (), unroll=False)

    def mlp_row(r, _):
        acc = par_v[12, :]
        for c in range(C):
            x = plsc.load_gather(cont_v, [jnp.full((LANES,), r * C + c,
                                                   dtype=jnp.int32)])
            acc = acc + ((x - par_v[4 + c, :]) * par_v[8 + c, :]) * par_v[c, :]
        blk_v[r * SLOTS + F, :] = acc
        return ()

    lax.fori_loop(0, BPW, mlp_row, (), unroll=False)

    pltpu.sync_copy(blk_v, out_hbm.at[wid])


@jax.jit
def _run(tab_flat, idx_w, cont_w, par):
    mesh = plsc.VectorSubcoreMesh(core_axis_name="c", subcore_axis_name="s",
                                  num_cores=NC, num_subcores=NS)
    kern = pl.kernel(
        _body,
        out_type=jax.ShapeDtypeStruct((NW, QPW, D), jnp.float32),
        mesh=mesh,
        scratch_types=[
            pltpu.VMEM((EPW,), jnp.int32),
            pltpu.VMEM((BPW * C,), jnp.float32),
            pltpu.VMEM((LANES, LANES), jnp.float32),
            pltpu.VMEM((SLOTS, BPW), jnp.int32),
            pltpu.VMEM((QPW, D), jnp.float32),
            pltpu.SemaphoreType.DMA,
        ],
        compiler_params=pltpu.CompilerParams(use_tc_tiling_on_sc=False,
                                             needs_layout_passes=False),
    )
    return kern(tab_flat, idx_w, cont_w, par)


def kernel(indices, cont, tables, means, variances, W, b):
    tab_flat = tables.reshape(F * V, D)
    idx_w = indices.astype(jnp.int32).reshape(NW, EPW)
    cont_w = cont.astype(jnp.float32).reshape(NW, BPW * C)

    inv_std = 1.0 / jnp.sqrt(variances.astype(jnp.float32))
    par = jnp.concatenate([
        W.astype(jnp.float32),
        jnp.broadcast_to(means.astype(jnp.float32)[:, None], (C, D)),
        jnp.broadcast_to(inv_std[:, None], (C, D)),
        jnp.broadcast_to(b.astype(jnp.float32)[None, :], (1, D)),
        jnp.zeros((3, D), jnp.float32),
    ], axis=0)

    out = _run(tab_flat, idx_w, cont_w, par)
    return out.reshape(B, SLOTS, D)


# speedup vs baseline: 2.0070x; 2.0070x over previous
import jax
import jax.numpy as jnp
from jax import lax
from jax.experimental import pallas as pl
from jax.experimental.pallas import tpu as pltpu, tpu_sc as plsc

B = 4096
F = 26
V = 100001
D = 16
C = 4

NC = 2
NS = 16
NW = NC * NS
BPW = B // NW
SLOTS = F + 1
QPW = SLOTS * BPW
LANES = 16


def _body(tab_hbm, idx_hbm, cont_hbm, par_hbm, out_hbm,
          idx_v, cont_v, par_v, blk_v, dsem):
    wid = lax.axis_index("c") * NS + lax.axis_index("s")
    base = wid * BPW

    pltpu.sync_copy(idx_hbm.at[:, pl.ds(base, BPW)], idx_v)
    pltpu.sync_copy(cont_hbm.at[:, pl.ds(base, BPW)], cont_v)
    pltpu.sync_copy(par_hbm, par_v)

    def gather_fire(f, _):
        pltpu.async_copy(
            tab_hbm.at[f].at[idx_v.at[f], :],
            blk_v.at[pl.ds(f * BPW, BPW)],
            dsem,
        )
        return ()

    lax.fori_loop(0, F, gather_fire, (), unroll=False)

    def gather_drain(f, _):
        pltpu.make_async_copy(
            tab_hbm.at[f].at[idx_v.at[f], :],
            blk_v.at[pl.ds(f * BPW, BPW)],
            dsem,
        ).wait()
        return ()

    lax.fori_loop(0, F, gather_drain, (), unroll=False)

    a_rows = [par_v[c, :] for c in range(C)]
    bias_row = par_v[C, :]

    def mlp_row(j, _):
        acc = bias_row
        for c in range(C):
            x = plsc.load_gather(cont_v, [jnp.full((LANES,), c, jnp.int32),
                                          jnp.full((LANES,), j, jnp.int32)])
            acc = acc + x * a_rows[c]
        blk_v[F * BPW + j, :] = acc
        return ()

    lax.fori_loop(0, BPW, mlp_row, (), unroll=False)

    pltpu.sync_copy(blk_v, out_hbm.at[wid])


@jax.jit
def _run(tab, idx_t, cont_t, par):
    mesh = plsc.VectorSubcoreMesh(core_axis_name="c", subcore_axis_name="s",
                                  num_cores=NC, num_subcores=NS)
    kern = pl.kernel(
        _body,
        out_type=jax.ShapeDtypeStruct((NW, QPW, D), jnp.float32),
        mesh=mesh,
        scratch_types=[
            pltpu.VMEM((F, BPW), jnp.int32),
            pltpu.VMEM((C, BPW), jnp.float32),
            pltpu.VMEM((C + 1, D), jnp.float32),
            pltpu.VMEM((QPW, D), jnp.float32),
            pltpu.SemaphoreType.DMA,
        ],
        compiler_params=pltpu.CompilerParams(use_tc_tiling_on_sc=False,
                                             needs_layout_passes=False),
    )
    return kern(tab, idx_t, cont_t, par)


def kernel(indices, cont, tables, means, variances, W, b):
    idx_t = indices.astype(jnp.int32).T
    cont_t = cont.astype(jnp.float32).T

    inv_std = 1.0 / jnp.sqrt(variances.astype(jnp.float32))
    A = W.astype(jnp.float32) * inv_std[:, None]
    bias = b.astype(jnp.float32) - (means.astype(jnp.float32) * inv_std) @ W
    par = jnp.concatenate([A, bias[None, :]], axis=0)

    out = _run(tables, idx_t, cont_t, par)
    return (out.reshape(NW, SLOTS, BPW, D)
               .transpose(0, 2, 1, 3)
               .reshape(B, SLOTS, D))
